# Initial kernel scaffold; baseline (speedup 1.0000x reference)
import jax
import jax.numpy as jnp
from jax.experimental import pallas as pl


def kernel(x):
    raise NotImplementedError("write your pallas kernel here")


# speedup vs baseline: 5.1474x; 5.1474x over previous
import jax
import jax.numpy as jnp
from jax.experimental import pallas as pl
from jax.experimental.pallas import tpu as pltpu

_C_BLK = 8
_H = 128
_W = 128


def _pool_body(x_ref, o_ref):
    v = x_ref[...]
    neg_inf = jnp.float32(-jnp.inf)
    for s in (1, 2, 4, 8, 16, 32, 64):
        pad = jnp.full((_C_BLK, s, _W), neg_inf, jnp.float32)
        shifted = jnp.concatenate([pad, v[:, : _H - s, :]], axis=1)
        v = jnp.maximum(v, shifted)
    for s in (1, 2, 4, 8, 16, 32, 64):
        pad = jnp.full((_C_BLK, _H, s), neg_inf, jnp.float32)
        shifted = jnp.concatenate([pad, v[:, :, : _W - s]], axis=2)
        v = jnp.maximum(v, shifted)
    o_ref[...] = v


def kernel(x):
    b, c, h, w = x.shape
    n = b * c
    xr = x.reshape(n, h, w)
    out = pl.pallas_call(
        _pool_body,
        grid=(n // _C_BLK,),
        in_specs=[pl.BlockSpec((_C_BLK, h, w), lambda i: (i, 0, 0))],
        out_specs=pl.BlockSpec((_C_BLK, h, w), lambda i: (i, 0, 0)),
        out_shape=jax.ShapeDtypeStruct((n, h, w), x.dtype),
        compiler_params=pltpu.CompilerParams(
            dimension_semantics=("parallel",),
        ),
    )(xr)
    return out.reshape(b, c, h, w)


# speedup vs baseline: 6.7553x; 1.3124x over previous
import jax
import jax.numpy as jnp
from jax.experimental import pallas as pl
from jax.experimental.pallas import tpu as pltpu

_C_BLK = 8
_H = 128
_W = 128


def _scan_rows(v):
    c = v.shape[0]
    neg_inf = jnp.float32(-jnp.inf)
    for s in (1, 2, 4, 8, 16, 32, 64):
        pad = jnp.full((c, s, v.shape[2]), neg_inf, jnp.float32)
        shifted = jnp.concatenate([pad, v[:, : v.shape[1] - s, :]], axis=1)
        v = jnp.maximum(v, shifted)
    return v


def _pool_body(x_ref, o_ref):
    v = _scan_rows(x_ref[...])
    vt = jnp.swapaxes(v, 1, 2)
    vt = _scan_rows(vt)
    o_ref[...] = jnp.swapaxes(vt, 1, 2)


def kernel(x):
    b, c, h, w = x.shape
    n = b * c
    xr = x.reshape(n, h, w)
    out = pl.pallas_call(
        _pool_body,
        grid=(n // _C_BLK,),
        in_specs=[pl.BlockSpec((_C_BLK, h, w), lambda i: (i, 0, 0))],
        out_specs=pl.BlockSpec((_C_BLK, h, w), lambda i: (i, 0, 0)),
        out_shape=jax.ShapeDtypeStruct((n, h, w), x.dtype),
        compiler_params=pltpu.CompilerParams(
            dimension_semantics=("parallel",),
        ),
    )(xr)
    return out.reshape(b, c, h, w)


# speedup vs baseline: 11.3817x; 1.6849x over previous
import jax
import jax.numpy as jnp
from jax.experimental import pallas as pl
from jax.experimental.pallas import tpu as pltpu

_C_BLK = 32
_H = 128
_W = 128


def _scan_rows(v):
    c = v.shape[0]
    neg_inf = jnp.float32(-jnp.inf)
    for s in (1, 2, 4, 8, 16, 32, 64):
        pad = jnp.full((c, s, v.shape[2]), neg_inf, jnp.float32)
        shifted = jnp.concatenate([pad, v[:, : v.shape[1] - s, :]], axis=1)
        v = jnp.maximum(v, shifted)
    return v


def _pool_body(x_ref, o_ref):
    v = _scan_rows(x_ref[...])
    vt = jnp.swapaxes(v, 1, 2)
    vt = _scan_rows(vt)
    o_ref[...] = jnp.swapaxes(vt, 1, 2)


def kernel(x):
    b, c, h, w = x.shape
    n = b * c
    xr = x.reshape(n, h, w)
    out = pl.pallas_call(
        _pool_body,
        grid=(n // _C_BLK,),
        in_specs=[pl.BlockSpec((_C_BLK, h, w), lambda i: (i, 0, 0))],
        out_specs=pl.BlockSpec((_C_BLK, h, w), lambda i: (i, 0, 0)),
        out_shape=jax.ShapeDtypeStruct((n, h, w), x.dtype),
        compiler_params=pltpu.CompilerParams(
            dimension_semantics=("parallel",),
        ),
    )(xr)
    return out.reshape(b, c, h, w)


# speedup vs baseline: 12.8141x; 1.1259x over previous
import jax
import jax.numpy as jnp
from jax.experimental import pallas as pl
from jax.experimental.pallas import tpu as pltpu

_C_BLK = 64
_H = 128
_W = 128


def _scan_rows(v):
    c = v.shape[0]
    neg_inf = jnp.float32(-jnp.inf)
    for s in (1, 2, 4, 8, 16, 32, 64):
        pad = jnp.full((c, s, v.shape[2]), neg_inf, jnp.float32)
        shifted = jnp.concatenate([pad, v[:, : v.shape[1] - s, :]], axis=1)
        v = jnp.maximum(v, shifted)
    return v


def _pool_body(x_ref, o_ref):
    v = _scan_rows(x_ref[...])
    vt = jnp.swapaxes(v, 1, 2)
    vt = _scan_rows(vt)
    o_ref[...] = jnp.swapaxes(vt, 1, 2)


def kernel(x):
    b, c, h, w = x.shape
    n = b * c
    xr = x.reshape(n, h, w)
    out = pl.pallas_call(
        _pool_body,
        grid=(n // _C_BLK,),
        in_specs=[pl.BlockSpec((_C_BLK, h, w), lambda i: (i, 0, 0))],
        out_specs=pl.BlockSpec((_C_BLK, h, w), lambda i: (i, 0, 0)),
        out_shape=jax.ShapeDtypeStruct((n, h, w), x.dtype),
        compiler_params=pltpu.CompilerParams(
            dimension_semantics=("parallel",),
        ),
    )(xr)
    return out.reshape(b, c, h, w)


# speedup vs baseline: 12.8985x; 1.0066x over previous
import jax
import jax.numpy as jnp
from jax.experimental import pallas as pl
from jax.experimental.pallas import tpu as pltpu

_C_BLK = 128
_H = 128
_W = 128


def _scan_rows(v):
    c = v.shape[0]
    neg_inf = jnp.float32(-jnp.inf)
    for s in (1, 2, 4, 8, 16, 32, 64):
        pad = jnp.full((c, s, v.shape[2]), neg_inf, jnp.float32)
        shifted = jnp.concatenate([pad, v[:, : v.shape[1] - s, :]], axis=1)
        v = jnp.maximum(v, shifted)
    return v


def _pool_body(x_ref, o_ref):
    v = _scan_rows(x_ref[...])
    vt = jnp.swapaxes(v, 1, 2)
    vt = _scan_rows(vt)
    o_ref[...] = jnp.swapaxes(vt, 1, 2)


def kernel(x):
    b, c, h, w = x.shape
    n = b * c
    xr = x.reshape(n, h, w)
    out = pl.pallas_call(
        _pool_body,
        grid=(n // _C_BLK,),
        in_specs=[pl.BlockSpec((_C_BLK, h, w), lambda i: (i, 0, 0))],
        out_specs=pl.BlockSpec((_C_BLK, h, w), lambda i: (i, 0, 0)),
        out_shape=jax.ShapeDtypeStruct((n, h, w), x.dtype),
        compiler_params=pltpu.CompilerParams(
            dimension_semantics=("parallel",),
        ),
    )(xr)
    return out.reshape(b, c, h, w)


# speedup vs baseline: 15.2969x; 1.1859x over previous
import jax
import jax.numpy as jnp
from jax.experimental import pallas as pl
from jax.experimental.pallas import tpu as pltpu

_C_BLK = 128
_H = 128
_W = 128


def _scan_rows(v):
    c, m, n = v.shape
    neg_inf = jnp.float32(-jnp.inf)
    for s in (1, 2, 4):
        pad = jnp.full((c, s, n), neg_inf, jnp.float32)
        shifted = jnp.concatenate([pad, v[:, : m - s, :]], axis=1)
        v = jnp.maximum(v, shifted)
    acc = v[:, 0:8, :]
    parts = [acc]
    for i in range(1, m // 8):
        acc = jnp.maximum(v[:, 8 * i : 8 * (i + 1), :], acc)
        parts.append(acc)
    return jnp.concatenate(parts, axis=1)


def _pool_body(x_ref, o_ref):
    v = _scan_rows(x_ref[...])
    vt = jnp.swapaxes(v, 1, 2)
    vt = _scan_rows(vt)
    o_ref[...] = jnp.swapaxes(vt, 1, 2)


def kernel(x):
    b, c, h, w = x.shape
    n = b * c
    blk = min(_C_BLK, n)
    xr = x.reshape(n, h, w)
    out = pl.pallas_call(
        _pool_body,
        grid=(n // blk,),
        in_specs=[pl.BlockSpec((blk, h, w), lambda i: (i, 0, 0))],
        out_specs=pl.BlockSpec((blk, h, w), lambda i: (i, 0, 0)),
        out_shape=jax.ShapeDtypeStruct((n, h, w), x.dtype),
        compiler_params=pltpu.CompilerParams(
            dimension_semantics=("parallel",),
        ),
    )(xr)
    return out.reshape(b, c, h, w)
